# baseline (speedup 1.0000x reference)
import dataclasses
import functools

import jax
import jax.numpy as jnp
from jax import lax
from jax.experimental import pallas as pl
from jax.experimental.pallas import tpu as pltpu
from jax.experimental.pallas import tpu_sc as plsc

N = 10000
F = 128
E = 320000
NPAD = 10240
NC = 2
NS = 16
NW = NC * NS
CH = 128
RPW = 80
EP = NW * RPW * CH
ROWS = EP // CH
RPW0 = 160
NSL = NPAD // NS

_BIG_NEG = -3.0e38


def _zero_vmem_2d(buf, nrows, ncols):
    @pl.loop(0, nrows)
    def _(r):
        for c in range(ncols // 16):
            buf[r, pl.ds(c * 16, 16)] = jnp.zeros((16,), jnp.float32)


def _deg_kernel(dst_hbm, w_hbm, deg_hbm, dst_b, w_b, zb, deg_sh, sem_c, sem_s):
    cid = lax.axis_index("c")
    sid = lax.axis_index("s")
    wid = cid * NS + sid

    pltpu.async_copy(dst_hbm.at[wid], dst_b, sem_c)
    pltpu.async_copy(w_hbm.at[wid], w_b, sem_c)

    @pl.loop(0, NSL // 16)
    def _(k):
        zb[pl.ds(k * 16, 16)] = jnp.zeros((16,), jnp.float32)

    pltpu.sync_copy(zb, deg_sh.at[pl.ds(sid * NSL, NSL)])
    pltpu.make_async_copy(dst_hbm.at[wid], dst_b, sem_c).wait()
    pltpu.make_async_copy(w_hbm.at[wid], w_b, sem_c).wait()
    plsc.subcore_barrier()

    @pl.loop(0, RPW // 4)
    def _(g):
        for j in range(4):
            r = g * 4 + j
            pltpu.async_copy(w_b.at[r], deg_sh.at[dst_b.at[r]], sem_s,
                             add=True)
        for j in range(4):
            r = g * 4 + j
            pltpu.make_async_copy(w_b.at[r], deg_sh.at[dst_b.at[r]],
                                  sem_s).wait()

    plsc.subcore_barrier()
    pltpu.sync_copy(deg_sh.at[pl.ds(sid * NSL, NSL)],
                    deg_hbm.at[pl.ds(cid * NPAD + sid * NSL, NSL)])


def _edge_kernel(edata_hbm, xwp_hbm, hp_hbm, eb, rb0, rb1, zb,
                 wrow, h_sh, se0, se1, se2, se3, sg0, sg1, ss0, ss1):
    cid = lax.axis_index("c")
    sid = lax.axis_index("s")
    rbs = (rb0, rb1)
    ses = (se0, se1, se2, se3)
    sgs = (sg0, sg1)

    base = sid * RPW0

    def e_start(t, k):
        pltpu.async_copy(edata_hbm.at[base + t], eb.at[pl.ds(k * 8, 8)],
                         ses[k])

    def e_wait(t, k):
        pltpu.make_async_copy(edata_hbm.at[base + t],
                              eb.at[pl.ds(k * 8, 8)], ses[k]).wait()

    def g_start(k, s):
        pltpu.async_copy(xwp_hbm.at[eb.at[k * 8]], rbs[s], sgs[s])

    def g_wait(k, s):
        pltpu.make_async_copy(xwp_hbm.at[eb.at[k * 8]], rbs[s],
                              sgs[s]).wait()

    def s_do(k, s):
        pltpu.sync_copy(rbs[s], h_sh.at[eb.at[k * 8 + 1]], add=True)

    def compute(k, s):
        rbuf = rbs[s]

        @pl.loop(0, CH // 16)
        def _(g):
            wrow[pl.ds(g * 16, 16)] = plsc.bitcast(
                eb[k * 8 + 2, pl.ds(g * 16, 16)], jnp.float32)

        @pl.loop(0, CH, step=4)
        def _(ev):
            for du in range(4):
                e = ev + du
                wspl = plsc.load_gather(wrow, [jnp.full((16,), e,
                                                        jnp.int32)])
                for f in range(F // 16):
                    sl = (e, pl.ds(f * 16, 16))
                    rbuf[sl] = rbuf[sl] * wspl

    def stage(t, nset, skip_next=False):
        k, s = t % 4, t % 2
        kn, sn = (t + 1) % 4, (t + 1) % 2
        if not skip_next:
            e_wait(t + 1, kn)
            g_start(kn, sn)
            e_start(jnp.minimum(t + 3, nset - 1), (t + 3) % 4)
        g_wait(k, s)
        compute(k, s)
        s_do(k, s)

    def run_pipeline(nset):
        e_start(0, 0)
        e_start(1, 1)
        e_start(2, 2)

        _zero_vmem_2d(zb, 20, F)
        for j in range(NSL // 20):
            pltpu.sync_copy(zb, h_sh.at[pl.ds(sid * NSL + j * 20, 20)])

        e_wait(0, 0)
        g_start(0, 0)
        plsc.subcore_barrier()
        stage(0, nset)
        stage(1, nset)
        stage(2, nset)

        @pl.loop(0, (nset - 4) // 4)
        def _(u):
            for q in range(4):
                t = 3 + 4 * u + q

                k, s = (3 + q) % 4, (3 + q) % 2
                kn, sn = (4 + q) % 4, (4 + q) % 2
                e_wait(t + 1, kn)
                g_start(kn, sn)
                e_start(jnp.minimum(t + 3, nset - 1), (2 + q) % 4)
                g_wait(k, s)
                compute(k, s)
                s_do(k, s)

        stage(nset - 1, nset, skip_next=True)
        e_wait(nset - 1, 0)
        e_wait(nset - 1, 1)

    @pl.when(cid == 0)
    def _():
        run_pipeline(RPW0)
        plsc.subcore_barrier()
        pltpu.sync_copy(h_sh.at[pl.ds(sid * NSL, NSL)],
                        hp_hbm.at[pl.ds(sid * NSL, NSL)])


def _dense_pre_kernel(xp_ref, xt_ref, p_ref, wih_ref, whh_ref, bih_ref,
                      bhh_ref, h0_ref, degp_ref, xwp_ref, dinv_ref,
                      s_ref, oh_ref):
    pvec = p_ref[...]
    inv_norm = jax.lax.rsqrt(jnp.sum(pvec * pvec))
    s = jnp.dot(pvec, xt_ref[...],
                preferred_element_type=jnp.float32)
    lane = lax.broadcasted_iota(jnp.int32, (1, NPAD), 1)
    s_ref[...] = jnp.where(lane < N, s, _BIG_NEG)

    def body(i, _):
        sv = s_ref[...]
        maxv = jnp.max(sv)
        m = jnp.min(jnp.where(sv == maxv, lane, jnp.int32(2 ** 30)))
        val = jnp.tanh(maxv * inv_norm)
        hit = lane == m
        oh_ref[pl.ds(i, 1), :] = jnp.where(hit, val, 0.0)
        s_ref[...] = jnp.where(hit, _BIG_NEG, sv)
        return 0

    lax.fori_loop(0, F, body, 0)

    x_tilde = jnp.dot(oh_ref[...], xp_ref[...],
                      preferred_element_type=jnp.float32)

    h0 = h0_ref[...]

    def gate(idx):
        wi = wih_ref[pl.ds(idx * F, F), :]
        wh = whh_ref[pl.ds(idx * F, F), :]
        gi = lax.dot_general(x_tilde, wi, (((1,), (1,)), ((), ())),
                             preferred_element_type=jnp.float32)
        gh = lax.dot_general(h0, wh, (((1,), (1,)), ((), ())),
                             preferred_element_type=jnp.float32)
        gi = gi + bih_ref[pl.ds(idx, 1), :]
        gh = gh + bhh_ref[pl.ds(idx, 1), :]
        return gi, gh

    i_r, h_r = gate(0)
    i_z, h_z = gate(1)
    i_n, h_n = gate(2)
    r = jax.nn.sigmoid(i_r + h_r)
    z = jax.nn.sigmoid(i_z + h_z)
    n = jnp.tanh(i_n + r * h_n)
    w_ev = (1.0 - z) * n + z * h0

    deg = (1.0 + degp_ref[pl.ds(0, NPAD), :]
           + degp_ref[pl.ds(NPAD, NPAD), :])
    dinv = jax.lax.rsqrt(deg)
    dinv_ref[...] = dinv

    xw = jnp.dot(xp_ref[...], w_ev,
                 preferred_element_type=jnp.float32)
    xwp_ref[...] = xw * dinv


def _final_kernel(hp_ref, xwp_ref, dinv_ref, wlin_ref, blin_ref, out_ref):
    acc = hp_ref[...] + xwp_ref[...]
    h = jnp.maximum(acc * dinv_ref[...], 0.0)
    out = lax.dot_general(h, wlin_ref[...], (((1,), (1,)), ((), ())),
                          preferred_element_type=jnp.float32)
    out_ref[...] = out[:N, :] + blin_ref[...]


def kernel(x, edge_index, edge_weight, p, W_ih, W_hh, b_ih, b_hh, h0,
           W_lin, b_lin):
    f32 = jnp.float32
    pad_i = jnp.zeros((EP - E,), jnp.int32)
    src = jnp.concatenate([edge_index[0], pad_i]).reshape(NW, RPW, CH)
    dst = jnp.concatenate([edge_index[1], pad_i]).reshape(NW, RPW, CH)
    wp = jnp.concatenate([edge_weight, jnp.zeros((EP - E,), f32)])
    w3d = wp.reshape(NW, RPW, CH)
    w_bits = lax.bitcast_convert_type(wp, jnp.int32).reshape(NW, RPW, CH)
    edata = jnp.concatenate(
        [jnp.stack([src, dst, w_bits], axis=2),
         jnp.zeros((NW, RPW, 5, CH), jnp.int32)],
        axis=2).reshape(ROWS, 8, CH)

    xp = jnp.zeros((NPAD, F), f32).at[:N, :].set(x)
    xt = xp.T

    vector_mesh = plsc.VectorSubcoreMesh(core_axis_name="c",
                                         subcore_axis_name="s")
    sc_params = pltpu.CompilerParams()
    if "needs_layout_passes" in pltpu.CompilerParams.__dataclass_fields__:
        sc_params = dataclasses.replace(sc_params, needs_layout_passes=False)

    deg_call = pl.kernel(
        _deg_kernel,
        out_type=jax.ShapeDtypeStruct((NC * NPAD,), f32),
        mesh=vector_mesh,
        scratch_types=[
            pltpu.VMEM((RPW, CH), jnp.int32),
            pltpu.VMEM((RPW, CH), f32),
            pltpu.VMEM((NSL,), f32),
            pltpu.VMEM_SHARED((NPAD,), f32),
            pltpu.SemaphoreType.DMA,
            pltpu.SemaphoreType.DMA,
        ],
        compiler_params=sc_params,
    )
    degp = deg_call(dst, w3d).reshape(NC * NPAD, 1)

    dense_call = pl.pallas_call(
        _dense_pre_kernel,
        out_shape=(jax.ShapeDtypeStruct((NPAD, F), f32),
                   jax.ShapeDtypeStruct((NPAD, 1), f32)),
        scratch_shapes=[pltpu.VMEM((1, NPAD), f32),
                        pltpu.VMEM((F, NPAD), f32)],
    )
    xwp, dinv = dense_call(xp, xt, p.reshape(1, F), W_ih, W_hh,
                           b_ih.reshape(3, F), b_hh.reshape(3, F), h0, degp)

    edge_call = pl.kernel(
        _edge_kernel,
        out_type=jax.ShapeDtypeStruct((NPAD, F), f32),
        mesh=vector_mesh,
        scratch_types=[
            pltpu.VMEM((32, CH), jnp.int32),
            pltpu.VMEM((CH, F), f32),
            pltpu.VMEM((CH, F), f32),
            pltpu.VMEM((20, F), f32),
            pltpu.VMEM((CH,), f32),
            pltpu.VMEM_SHARED((NPAD, F), f32),
        ] + [pltpu.SemaphoreType.DMA] * 8,
        compiler_params=sc_params,
    )
    hp = edge_call(edata, xwp)

    final_call = pl.pallas_call(
        _final_kernel,
        out_shape=jax.ShapeDtypeStruct((N, F), f32),
    )
    return final_call(hp, xwp, dinv, W_lin, b_lin.reshape(1, F))


# speedup vs baseline: 1.2795x; 1.2795x over previous
import dataclasses
import functools

import jax
import jax.numpy as jnp
from jax import lax
from jax.experimental import pallas as pl
from jax.experimental.pallas import tpu as pltpu
from jax.experimental.pallas import tpu_sc as plsc

N = 10000
F = 128
E = 320000
NPAD = 10240
NC = 2
NS = 16
NW = NC * NS
CH = 128
RPW = 80
EP = NW * RPW * CH
ROWS = EP // CH
RPW0 = 140
RPW1 = 2 * RPW - RPW0
NSL = NPAD // NS

_BIG_NEG = -3.0e38


def _zero_vmem_2d(buf, nrows, ncols):
    @pl.loop(0, nrows)
    def _(r):
        for c in range(ncols // 16):
            buf[r, pl.ds(c * 16, 16)] = jnp.zeros((16,), jnp.float32)


def _deg_kernel(dst_hbm, w_hbm, deg_hbm, dst_b, w_b, zb, deg_sh, sem_c, sem_s):
    cid = lax.axis_index("c")
    sid = lax.axis_index("s")
    wid = cid * NS + sid

    pltpu.async_copy(dst_hbm.at[wid], dst_b, sem_c)
    pltpu.async_copy(w_hbm.at[wid], w_b, sem_c)

    @pl.loop(0, NSL // 16)
    def _(k):
        zb[pl.ds(k * 16, 16)] = jnp.zeros((16,), jnp.float32)

    pltpu.sync_copy(zb, deg_sh.at[pl.ds(sid * NSL, NSL)])
    pltpu.make_async_copy(dst_hbm.at[wid], dst_b, sem_c).wait()
    pltpu.make_async_copy(w_hbm.at[wid], w_b, sem_c).wait()
    plsc.subcore_barrier()

    @pl.loop(0, RPW // 4)
    def _(g):
        for j in range(4):
            r = g * 4 + j
            pltpu.async_copy(w_b.at[r], deg_sh.at[dst_b.at[r]], sem_s,
                             add=True)
        for j in range(4):
            r = g * 4 + j
            pltpu.make_async_copy(w_b.at[r], deg_sh.at[dst_b.at[r]],
                                  sem_s).wait()

    plsc.subcore_barrier()
    pltpu.sync_copy(deg_sh.at[pl.ds(sid * NSL, NSL)],
                    deg_hbm.at[pl.ds(cid * NPAD + sid * NSL, NSL)])


def _edge_kernel(edata_hbm, xwp_hbm, hp_hbm, eb, rb0, rb1, zb,
                 wrow, h_sh, se0, se1, se2, se3, sg0, sg1, ss0, ss1):
    cid = lax.axis_index("c")
    sid = lax.axis_index("s")
    rbs = (rb0, rb1)
    ses = (se0, se1, se2, se3)
    sgs = (sg0, sg1)

    base = jnp.where(cid == 0, sid * RPW0, NS * RPW0 + sid * RPW1)

    def e_start(t, k):
        pltpu.async_copy(edata_hbm.at[base + t], eb.at[pl.ds(k * 8, 8)],
                         ses[k])

    def e_wait(t, k):
        pltpu.make_async_copy(edata_hbm.at[base + t],
                              eb.at[pl.ds(k * 8, 8)], ses[k]).wait()

    def g_start(k, s):
        pltpu.async_copy(xwp_hbm.at[eb.at[k * 8]], rbs[s], sgs[s])

    def g_wait(k, s):
        pltpu.make_async_copy(xwp_hbm.at[eb.at[k * 8]], rbs[s],
                              sgs[s]).wait()

    def s_do(k, s):
        pltpu.sync_copy(rbs[s], h_sh.at[eb.at[k * 8 + 1]], add=True)

    def compute(k, s):
        rbuf = rbs[s]

        @pl.loop(0, CH // 16)
        def _(g):
            wrow[pl.ds(g * 16, 16)] = plsc.bitcast(
                eb[k * 8 + 2, pl.ds(g * 16, 16)], jnp.float32)

        @pl.loop(0, CH, step=4)
        def _(ev):
            for du in range(4):
                e = ev + du
                wspl = plsc.load_gather(wrow, [jnp.full((16,), e,
                                                        jnp.int32)])
                for f in range(F // 16):
                    sl = (e, pl.ds(f * 16, 16))
                    rbuf[sl] = rbuf[sl] * wspl

    def stage(t, nset, skip_next=False):
        k, s = t % 4, t % 2
        kn, sn = (t + 1) % 4, (t + 1) % 2
        if not skip_next:
            e_wait(t + 1, kn)
            g_start(kn, sn)
            e_start(jnp.minimum(t + 3, nset - 1), (t + 3) % 4)
        g_wait(k, s)
        compute(k, s)
        s_do(k, s)

    def run_pipeline(nset):
        e_start(0, 0)
        e_start(1, 1)
        e_start(2, 2)

        _zero_vmem_2d(zb, 20, F)
        for j in range(NSL // 20):
            pltpu.sync_copy(zb, h_sh.at[pl.ds(sid * NSL + j * 20, 20)])

        e_wait(0, 0)
        g_start(0, 0)
        plsc.subcore_barrier()
        stage(0, nset)
        stage(1, nset)
        stage(2, nset)

        @pl.loop(0, (nset - 4) // 4)
        def _(u):
            for q in range(4):
                t = 3 + 4 * u + q

                k, s = (3 + q) % 4, (3 + q) % 2
                kn, sn = (4 + q) % 4, (4 + q) % 2
                e_wait(t + 1, kn)
                g_start(kn, sn)
                e_start(jnp.minimum(t + 3, nset - 1), (2 + q) % 4)
                g_wait(k, s)
                compute(k, s)
                s_do(k, s)

        stage(nset - 1, nset, skip_next=True)
        e_wait(nset - 1, 0)
        e_wait(nset - 1, 1)

    @pl.when(cid == 0)
    def _():
        run_pipeline(RPW0)

    @pl.when(cid == 1)
    def _():
        run_pipeline(RPW1)

    plsc.subcore_barrier()
    pltpu.sync_copy(h_sh.at[pl.ds(sid * NSL, NSL)],
                    hp_hbm.at[pl.ds(cid * NPAD + sid * NSL, NSL)])


def _dense_pre_kernel(xp_ref, xt_ref, p_ref, wih_ref, whh_ref, bih_ref,
                      bhh_ref, h0_ref, degp_ref, xwp_ref, dinv_ref,
                      s_ref, oh_ref):
    pvec = p_ref[...]
    inv_norm = jax.lax.rsqrt(jnp.sum(pvec * pvec))
    s = jnp.dot(pvec, xt_ref[...],
                preferred_element_type=jnp.float32)
    lane = lax.broadcasted_iota(jnp.int32, (1, NPAD), 1)
    s_ref[...] = jnp.where(lane < N, s, _BIG_NEG)

    def body(i, _):
        sv = s_ref[...]
        maxv = jnp.max(sv)
        m = jnp.min(jnp.where(sv == maxv, lane, jnp.int32(2 ** 30)))
        val = jnp.tanh(maxv * inv_norm)
        hit = lane == m
        oh_ref[pl.ds(i, 1), :] = jnp.where(hit, val, 0.0)
        s_ref[...] = jnp.where(hit, _BIG_NEG, sv)
        return 0

    lax.fori_loop(0, F, body, 0)

    x_tilde = jnp.dot(oh_ref[...], xp_ref[...],
                      preferred_element_type=jnp.float32)

    h0 = h0_ref[...]

    def gate(idx):
        wi = wih_ref[pl.ds(idx * F, F), :]
        wh = whh_ref[pl.ds(idx * F, F), :]
        gi = lax.dot_general(x_tilde, wi, (((1,), (1,)), ((), ())),
                             preferred_element_type=jnp.float32)
        gh = lax.dot_general(h0, wh, (((1,), (1,)), ((), ())),
                             preferred_element_type=jnp.float32)
        gi = gi + bih_ref[pl.ds(idx, 1), :]
        gh = gh + bhh_ref[pl.ds(idx, 1), :]
        return gi, gh

    i_r, h_r = gate(0)
    i_z, h_z = gate(1)
    i_n, h_n = gate(2)
    r = jax.nn.sigmoid(i_r + h_r)
    z = jax.nn.sigmoid(i_z + h_z)
    n = jnp.tanh(i_n + r * h_n)
    w_ev = (1.0 - z) * n + z * h0

    deg = (1.0 + degp_ref[pl.ds(0, NPAD), :]
           + degp_ref[pl.ds(NPAD, NPAD), :])
    dinv = jax.lax.rsqrt(deg)
    dinv_ref[...] = dinv

    xw = jnp.dot(xp_ref[...], w_ev,
                 preferred_element_type=jnp.float32)
    xwp_ref[...] = xw * dinv


def _final_kernel(hp_ref, xwp_ref, dinv_ref, wlin_ref, blin_ref, out_ref):
    acc = (hp_ref[pl.ds(0, NPAD), :] + hp_ref[pl.ds(NPAD, NPAD), :]
           + xwp_ref[...])
    h = jnp.maximum(acc * dinv_ref[...], 0.0)
    out = lax.dot_general(h, wlin_ref[...], (((1,), (1,)), ((), ())),
                          preferred_element_type=jnp.float32)
    out_ref[...] = out[:N, :] + blin_ref[...]


def kernel(x, edge_index, edge_weight, p, W_ih, W_hh, b_ih, b_hh, h0,
           W_lin, b_lin):
    f32 = jnp.float32
    pad_i = jnp.zeros((EP - E,), jnp.int32)
    src = jnp.concatenate([edge_index[0], pad_i]).reshape(NW, RPW, CH)
    dst = jnp.concatenate([edge_index[1], pad_i]).reshape(NW, RPW, CH)
    wp = jnp.concatenate([edge_weight, jnp.zeros((EP - E,), f32)])
    w3d = wp.reshape(NW, RPW, CH)
    w_bits = lax.bitcast_convert_type(wp, jnp.int32).reshape(NW, RPW, CH)
    edata = jnp.concatenate(
        [jnp.stack([src, dst, w_bits], axis=2),
         jnp.zeros((NW, RPW, 5, CH), jnp.int32)],
        axis=2).reshape(ROWS, 8, CH)

    xp = jnp.zeros((NPAD, F), f32).at[:N, :].set(x)
    xt = xp.T

    vector_mesh = plsc.VectorSubcoreMesh(core_axis_name="c",
                                         subcore_axis_name="s")
    sc_params = pltpu.CompilerParams()
    if "needs_layout_passes" in pltpu.CompilerParams.__dataclass_fields__:
        sc_params = dataclasses.replace(sc_params, needs_layout_passes=False)

    deg_call = pl.kernel(
        _deg_kernel,
        out_type=jax.ShapeDtypeStruct((NC * NPAD,), f32),
        mesh=vector_mesh,
        scratch_types=[
            pltpu.VMEM((RPW, CH), jnp.int32),
            pltpu.VMEM((RPW, CH), f32),
            pltpu.VMEM((NSL,), f32),
            pltpu.VMEM_SHARED((NPAD,), f32),
            pltpu.SemaphoreType.DMA,
            pltpu.SemaphoreType.DMA,
        ],
        compiler_params=sc_params,
    )
    degp = deg_call(dst, w3d).reshape(NC * NPAD, 1)

    dense_call = pl.pallas_call(
        _dense_pre_kernel,
        out_shape=(jax.ShapeDtypeStruct((NPAD, F), f32),
                   jax.ShapeDtypeStruct((NPAD, 1), f32)),
        scratch_shapes=[pltpu.VMEM((1, NPAD), f32),
                        pltpu.VMEM((F, NPAD), f32)],
    )
    xwp, dinv = dense_call(xp, xt, p.reshape(1, F), W_ih, W_hh,
                           b_ih.reshape(3, F), b_hh.reshape(3, F), h0, degp)

    edge_call = pl.kernel(
        _edge_kernel,
        out_type=jax.ShapeDtypeStruct((NC * NPAD, F), f32),
        mesh=vector_mesh,
        scratch_types=[
            pltpu.VMEM((32, CH), jnp.int32),
            pltpu.VMEM((CH, F), f32),
            pltpu.VMEM((CH, F), f32),
            pltpu.VMEM((20, F), f32),
            pltpu.VMEM((CH,), f32),
            pltpu.VMEM_SHARED((NPAD, F), f32),
        ] + [pltpu.SemaphoreType.DMA] * 8,
        compiler_params=sc_params,
    )
    hp = edge_call(edata, xwp)

    final_call = pl.pallas_call(
        _final_kernel,
        out_shape=jax.ShapeDtypeStruct((N, F), f32),
    )
    return final_call(hp, xwp, dinv, W_lin, b_lin.reshape(1, F))
